# baseline (speedup 1.0000x reference)
import functools

import jax
import jax.numpy as jnp
from jax import lax
from jax.experimental import pallas as pl
from jax.experimental.pallas import tpu as pltpu
from jax.experimental.pallas import tpu_sc as plsc

B = 8
R = 1000
A = 4
T = 20
F = 32

NC = 2
NS = 16
NW = NC * NS

PAIRS = B * T
PPW = PAIRS // NW
NRCH = R // 16 + 1
L = 16


def _make_gather():
    mesh = plsc.VectorSubcoreMesh(core_axis_name="c", subcore_axis_name="s")

    @functools.partial(
        pl.kernel,
        mesh=mesh,
        compiler_params=pltpu.CompilerParams(
            use_tc_tiling_on_sc=False, needs_layout_passes=False),
        out_type=jax.ShapeDtypeStruct((B, A, T, F, R), jnp.float32),
        scratch_types=[
            pltpu.VMEM((A, R), jnp.int32),
            pltpu.VMEM((F, R), jnp.float32),
            pltpu.VMEM((F, R), jnp.float32),
            pltpu.VMEM((F, R), jnp.float32),
            pltpu.SemaphoreType.DMA,
            pltpu.SemaphoreType.DMA,
        ],
    )
    def gather_kernel(xt_hbm, armt_hbm, out_hbm, perms, in_slab,
                      out_a, out_b, sem_a, sem_b):
        out_slabs = (out_a, out_b)
        sems = (sem_a, sem_b)
        wid = lax.axis_index("s") * NC + lax.axis_index("c")
        pair0 = wid * PPW
        b = pair0 // T
        pltpu.sync_copy(armt_hbm.at[pl.ds(b * A, A)], perms)

        def slab_body(p, carry):
            t = pair0 % T + p
            pltpu.sync_copy(xt_hbm.at[b, t], in_slab)
            scat = [None, None]
            for a in range(A):
                k = a % 2
                out_slab = out_slabs[k]
                if scat[k] is not None:
                    scat[k].wait()

                @plsc.parallel_loop(0, (R // L) * L, step=L, unroll=2)
                def rchunk(rb, a=a, out_slab=out_slab):
                    rb = pl.multiple_of(rb, L)
                    pv = perms[a, pl.ds(rb, L)]
                    for f in range(F):
                        row = jnp.full((L,), f, jnp.int32)
                        vals = plsc.load_gather(in_slab, [row, pv])
                        out_slab[f, pl.ds(rb, L)] = vals
                pv = perms[a, pl.ds(R - L, L)]
                for f in range(F):
                    row = jnp.full((L,), f, jnp.int32)
                    vals = plsc.load_gather(in_slab, [row, pv])
                    out_slab[f, pl.ds(R - L, L)] = vals
                scat[k] = pltpu.async_copy(
                    out_slab, out_hbm.at[b, a, t], sems[k])
            scat[0].wait()
            scat[1].wait()
            return carry

        lax.fori_loop(0, PPW, slab_body, 0)

    return gather_kernel


def kernel(x, arm):
    xt = jnp.transpose(x, (0, 2, 3, 1))
    armt = jnp.transpose(arm, (0, 2, 1)).reshape(B * A, R)
    outp = _make_gather()(xt, armt)
    return jnp.transpose(outp, (0, 4, 1, 2, 3))


# speedup vs baseline: 1.0923x; 1.0923x over previous
import functools

import jax
import jax.numpy as jnp
from jax import lax
from jax.experimental import pallas as pl
from jax.experimental.pallas import tpu as pltpu
from jax.experimental.pallas import tpu_sc as plsc

B = 8
R = 1000
A = 4
T = 20
F = 32

NC = 2
NS = 16
NW = NC * NS

PAIRS = B * T
PPW = PAIRS // NW
NRCH = R // 16 + 1
L = 16


def _make_gather():
    mesh = plsc.VectorSubcoreMesh(core_axis_name="c", subcore_axis_name="s")

    @functools.partial(
        pl.kernel,
        mesh=mesh,
        compiler_params=pltpu.CompilerParams(
            use_tc_tiling_on_sc=False, needs_layout_passes=False),
        out_type=jax.ShapeDtypeStruct((B, A, T, F, R), jnp.float32),
        scratch_types=[
            pltpu.VMEM((A, R), jnp.int32),
            pltpu.VMEM((F, R), jnp.float32),
            pltpu.VMEM((F, R), jnp.float32),
            pltpu.VMEM((F, R), jnp.float32),
            pltpu.SemaphoreType.DMA,
            pltpu.SemaphoreType.DMA,
        ],
    )
    def gather_kernel(xt_hbm, armt_hbm, out_hbm, perms, in_slab,
                      out_a, out_b, sem_a, sem_b):
        out_slabs = (out_a, out_b)
        sems = (sem_a, sem_b)
        wid = lax.axis_index("s") * NC + lax.axis_index("c")
        pair0 = wid * PPW
        b = pair0 // T
        pltpu.sync_copy(armt_hbm.at[pl.ds(b * A, A)], perms)

        def slab_body(p, carry):
            t = pair0 % T + p
            pltpu.sync_copy(xt_hbm.at[b, t], in_slab)
            scat = [None, None]
            for a in range(A):
                k = a % 2
                out_slab = out_slabs[k]
                if scat[k] is not None:
                    scat[k].wait()

                @plsc.parallel_loop(0, (R // L) * L, step=L)
                def rchunk(rb, a=a, out_slab=out_slab):
                    rb = pl.multiple_of(rb, L)
                    pv = perms[a, pl.ds(rb, L)]
                    for f in range(F):
                        row = jnp.full((L,), f, jnp.int32)
                        vals = plsc.load_gather(in_slab, [row, pv])
                        out_slab[f, pl.ds(rb, L)] = vals
                pv = perms[a, pl.ds(R - L, L)]
                for f in range(F):
                    row = jnp.full((L,), f, jnp.int32)
                    vals = plsc.load_gather(in_slab, [row, pv])
                    out_slab[f, pl.ds(R - L, L)] = vals
                scat[k] = pltpu.async_copy(
                    out_slab, out_hbm.at[b, a, t], sems[k])
            scat[0].wait()
            scat[1].wait()
            return carry

        lax.fori_loop(0, PPW, slab_body, 0)

    return gather_kernel


def kernel(x, arm):
    xt = jnp.transpose(x, (0, 2, 3, 1))
    armt = jnp.transpose(arm, (0, 2, 1)).reshape(B * A, R)
    outp = _make_gather()(xt, armt)
    return jnp.transpose(outp, (0, 4, 1, 2, 3))
